# baseline (speedup 1.0000x reference)
import functools
import jax
import jax.numpy as jnp
from jax import lax
from jax.experimental import pallas as pl
from jax.experimental.pallas import tpu as pltpu
from jax.experimental.pallas import tpu_sc as plsc

NC = 2
NS = 16
NW = NC * NS
CH = 128


def _sc_mesh():
    return plsc.VectorSubcoreMesh(core_axis_name="c", subcore_axis_name="s")


def _make_deg_kernel(e_pad, n_pad):
    j_per_w = e_pad // (NW * CH)
    rpt = n_pad // NS
    dw = 16

    @functools.partial(
        pl.kernel,
        out_type=jax.ShapeDtypeStruct((NC, n_pad), jnp.float32),
        mesh=_sc_mesh(),
        scratch_types=[
            pltpu.VMEM((j_per_w, CH), jnp.int32),
            pltpu.VMEM((CH,), jnp.float32),
            pltpu.VMEM((rpt,), jnp.float32),
            pltpu.VMEM_SHARED((n_pad,), jnp.float32),
        ],
    )
    def deg_kernel(dst_hbm, out_hbm, dst_v, ones_v, zbuf, deg_sh):
        c = lax.axis_index("c")
        s = lax.axis_index("s")
        w = c * NS + s

        ones = jnp.ones((16,), jnp.float32)
        zeros = jnp.zeros((16,), jnp.float32)

        for i in range(CH // 16):
            ones_v[pl.ds(i * 16, 16)] = ones

        def fill_zero(i, _):
            zbuf[pl.ds(i * 16, 16)] = zeros
            return 0
        lax.fori_loop(0, rpt // 16, fill_zero, 0)

        pltpu.sync_copy(zbuf, deg_sh.at[pl.ds(s * rpt, rpt)])
        pltpu.sync_copy(dst_hbm.at[pl.ds(w * j_per_w, j_per_w)], dst_v)
        plsc.subcore_barrier()

        def body(j, _):
            pltpu.sync_copy(ones_v, deg_sh.at[dst_v.at[j]], add=True)
            return 0
        lax.fori_loop(0, j_per_w, body, 0)

        plsc.subcore_barrier()
        pltpu.sync_copy(deg_sh.at[pl.ds(s * rpt, rpt)],
                        out_hbm.at[c, pl.ds(s * rpt, rpt)])

    return deg_kernel


def _make_spmm_kernel(e_pad, n_pad, d):
    half = n_pad // NC
    j_per_w = e_pad // (NS * CH)
    rpt = half // NS
    zr = 64

    K = 2
    n_grp = j_per_w // K

    @functools.partial(
        pl.kernel,
        out_type=jax.ShapeDtypeStruct((NC, half, d), jnp.float32),
        mesh=_sc_mesh(),
        scratch_types=[
            pltpu.VMEM((j_per_w, CH), jnp.int32),
            pltpu.VMEM((j_per_w, CH), jnp.int32),
            [pltpu.VMEM((CH, d), jnp.float32) for _ in range(K)],
            [pltpu.SemaphoreType.DMA for _ in range(K)],
            [pltpu.SemaphoreType.DMA for _ in range(K)],
            pltpu.VMEM_SHARED((half + 8, d), jnp.float32),
        ],
    )
    def spmm_kernel(g_hbm, src_hbm, dst_hbm, out_hbm,
                    src_v, dst_v, rows, gsem, ssem, acc_sh):
        c = lax.axis_index("c")
        s = lax.axis_index("s")

        def zrow(i, _):
            for cc in range(d // 16):
                rows[0][i, pl.ds(cc * 16, 16)] = jnp.zeros((16,), jnp.float32)
            return 0
        lax.fori_loop(0, CH, zrow, 0)
        nz = rpt // CH
        for k in range(nz):
            pltpu.sync_copy(rows[0], acc_sh.at[pl.ds(s * rpt + k * CH, CH)])
        if rpt % CH:
            pltpu.sync_copy(rows[0].at[pl.ds(0, rpt % CH)],
                            acc_sh.at[pl.ds(s * rpt + nz * CH, rpt % CH)])
        plsc.subcore_barrier()

        pltpu.sync_copy(src_hbm.at[pl.ds(s * j_per_w, j_per_w)], src_v)
        pltpu.sync_copy(dst_hbm.at[c, pl.ds(s * j_per_w, j_per_w)], dst_v)

        for r in range(K):
            pltpu.async_copy(g_hbm.at[src_v.at[r]], rows[r], gsem[r])

        def group(g, _):
            base = g * K
            for r in range(K):
                pltpu.make_async_copy(g_hbm.at[src_v.at[base + r]],
                                      rows[r], gsem[r]).wait()
                pltpu.async_copy(rows[r], acc_sh.at[dst_v.at[base + r]],
                                 ssem[r], add=True)

            @pl.when(g + 1 < n_grp)
            def _():
                for r in range(K):
                    pltpu.make_async_copy(
                        rows[r], acc_sh.at[dst_v.at[base + r]],
                        ssem[r]).wait()
                    pltpu.async_copy(g_hbm.at[src_v.at[base + K + r]],
                                     rows[r], gsem[r])
            return 0

        lax.fori_loop(0, n_grp, group, 0)
        last = (n_grp - 1) * K
        for r in range(K):
            pltpu.make_async_copy(rows[r], acc_sh.at[dst_v.at[last + r]],
                                  ssem[r]).wait()

        plsc.subcore_barrier()
        pltpu.sync_copy(acc_sh.at[pl.ds(s * rpt, rpt)],
                        out_hbm.at[c, pl.ds(s * rpt, rpt)])

    return spmm_kernel


def _tc1_body(cnt_ref, x_ref, w_ref, dinv_ref, g_ref):
    deg = cnt_ref[:, 0:1] + cnt_ref[:, 1:2] + 1.0
    dinv = lax.rsqrt(deg)
    dinv_ref[...] = dinv
    h = jnp.dot(x_ref[...], w_ref[...], preferred_element_type=jnp.float32)
    g_ref[...] = h * dinv


def _tc2_body(acc_ref, g1_ref, dinv_ref, b_ref, w_ref, g2_ref):
    a = acc_ref[...] + g1_ref[...]
    dinv = dinv_ref[...]
    h = jnp.maximum(dinv * a + b_ref[...], 0.0)
    g2_ref[...] = dinv * jnp.dot(h, w_ref[...],
                                 preferred_element_type=jnp.float32)


def _tc3_body(acc_ref, g2_ref, dinv_ref, b_ref, out_ref):
    a = acc_ref[...] + g2_ref[...]
    out_ref[...] = dinv_ref[...] * a + b_ref[...]


def kernel(x, edge_index, edge_attr, W1, b1, W2, b2):
    n, d = x.shape
    e = edge_index.shape[1]

    n_pad = (n + 1 + 255) // 256 * 256
    half = n_pad // NC
    e_pad = (e + NS * CH * 8 - 1) // (NS * CH * 8) * (NS * CH * 8)

    src = edge_index[0]
    dst = edge_index[1]
    pad = e_pad - e
    src_p = jnp.concatenate([src, jnp.full((pad,), n, jnp.int32)])
    dst_p = jnp.concatenate([dst, jnp.full((pad,), n, jnp.int32)])
    src2d = src_p.reshape(e_pad // CH, CH)
    dst2d = dst_p.reshape(e_pad // CH, CH)
    dst_cores = []
    for c in range(NC):
        lo = c * half
        owned = (dst_p >= lo) & (dst_p < lo + half)
        dst_cores.append(jnp.where(owned, dst_p - lo, half))
    dst3d = jnp.stack(dst_cores).reshape(NC, e_pad // CH, CH)

    x_pad = jnp.zeros((n_pad, d), x.dtype).at[:n].set(x)
    b1r = b1.reshape(1, d)
    b2r = b2.reshape(1, d)

    deg_k = _make_deg_kernel(e_pad, n_pad)
    spmm_k = _make_spmm_kernel(e_pad, n_pad, d)

    BISECT_JNP_DEG = False
    if BISECT_JNP_DEG:
        cnt0 = jnp.zeros((n_pad,), jnp.float32).at[dst].add(1.0)
        cnt_t = jnp.stack([cnt0, jnp.zeros_like(cnt0)], axis=1)
    else:
        cnt = deg_k(dst2d)
        cnt_t = cnt.T

    r = 1280
    grid = n_pad // r
    row_spec = pl.BlockSpec((r, d), lambda i: (i, 0))
    col_spec = pl.BlockSpec((r, 1), lambda i: (i, 0))
    full_spec = pl.BlockSpec((d, d), lambda i: (0, 0))
    bias_spec = pl.BlockSpec((1, d), lambda i: (0, 0))

    dinv, g1 = pl.pallas_call(
        _tc1_body,
        grid=grid,
        in_specs=[pl.BlockSpec((r, NC), lambda i: (i, 0)),
                  row_spec, full_spec],
        out_specs=[col_spec, row_spec],
        out_shape=[jax.ShapeDtypeStruct((n_pad, 1), jnp.float32),
                   jax.ShapeDtypeStruct((n_pad, d), jnp.float32)],
    )(cnt_t, x_pad, W1)

    acc1 = spmm_k(g1, src2d, dst3d).reshape(n_pad, d)

    g2 = pl.pallas_call(
        _tc2_body,
        grid=grid,
        in_specs=[row_spec, row_spec, col_spec, bias_spec, full_spec],
        out_specs=row_spec,
        out_shape=jax.ShapeDtypeStruct((n_pad, d), jnp.float32),
    )(acc1, g1, dinv, b1r, W2)

    acc2 = spmm_k(g2, src2d, dst3d).reshape(n_pad, d)

    out = pl.pallas_call(
        _tc3_body,
        grid=grid,
        in_specs=[row_spec, row_spec, col_spec, bias_spec],
        out_specs=row_spec,
        out_shape=jax.ShapeDtypeStruct((n_pad, d), jnp.float32),
    )(acc2, g2, dinv, b2r)

    return out[:n]


# speedup vs baseline: 1.2209x; 1.2209x over previous
import functools
import jax
import jax.numpy as jnp
from jax import lax
from jax.experimental import pallas as pl
from jax.experimental.pallas import tpu as pltpu
from jax.experimental.pallas import tpu_sc as plsc

NC = 2
NS = 16
NW = NC * NS
CH = 128


def _sc_mesh():
    return plsc.VectorSubcoreMesh(core_axis_name="c", subcore_axis_name="s")


def _make_deg_kernel(e_pad, n_pad):
    j_per_w = e_pad // (NW * CH)
    rpt = n_pad // NS
    dw = 16

    @functools.partial(
        pl.kernel,
        out_type=jax.ShapeDtypeStruct((NC, n_pad), jnp.float32),
        mesh=_sc_mesh(),
        scratch_types=[
            pltpu.VMEM((j_per_w, CH), jnp.int32),
            pltpu.VMEM((CH,), jnp.float32),
            pltpu.VMEM((rpt,), jnp.float32),
            pltpu.VMEM_SHARED((n_pad,), jnp.float32),
        ],
    )
    def deg_kernel(dst_hbm, out_hbm, dst_v, ones_v, zbuf, deg_sh):
        c = lax.axis_index("c")
        s = lax.axis_index("s")
        w = c * NS + s

        ones = jnp.ones((16,), jnp.float32)
        zeros = jnp.zeros((16,), jnp.float32)

        for i in range(CH // 16):
            ones_v[pl.ds(i * 16, 16)] = ones

        def fill_zero(i, _):
            zbuf[pl.ds(i * 16, 16)] = zeros
            return 0
        lax.fori_loop(0, rpt // 16, fill_zero, 0)

        pltpu.sync_copy(zbuf, deg_sh.at[pl.ds(s * rpt, rpt)])
        pltpu.sync_copy(dst_hbm.at[pl.ds(w * j_per_w, j_per_w)], dst_v)
        plsc.subcore_barrier()

        def body(j, _):
            pltpu.sync_copy(ones_v, deg_sh.at[dst_v.at[j]], add=True)
            return 0
        lax.fori_loop(0, j_per_w, body, 0)

        plsc.subcore_barrier()
        pltpu.sync_copy(deg_sh.at[pl.ds(s * rpt, rpt)],
                        out_hbm.at[c, pl.ds(s * rpt, rpt)])

    return deg_kernel


def _make_part_kernel(e_pad, n_pad, n_real):
    half = n_pad // NC
    jw = e_pad // (NS * CH)
    epw = e_pad // NS
    cap = epw + 16

    @functools.partial(
        pl.kernel,
        out_type=[
            jax.ShapeDtypeStruct((NC, NS * epw), jnp.int32),
            jax.ShapeDtypeStruct((NC, NS * epw), jnp.int32),
            jax.ShapeDtypeStruct((NC, NS, 16), jnp.int32),
        ],
        mesh=_sc_mesh(),
        scratch_types=[
            pltpu.VMEM((epw,), jnp.int32),
            pltpu.VMEM((epw,), jnp.int32),
            pltpu.VMEM((cap,), jnp.int32),
            pltpu.VMEM((16,), jnp.int32),
        ],
    )
    def part_kernel(src_hbm, dst_hbm, osrc_hbm, odst_hbm, ocnt_hbm,
                    src_v, dst_v, cpk_v, cnt_v):
        c = lax.axis_index("c")
        s = lax.axis_index("s")
        lo = c * half

        pltpu.sync_copy(src_hbm.at[pl.ds(s * epw, epw)], src_v)
        pltpu.sync_copy(dst_hbm.at[pl.ds(s * epw, epw)], dst_v)

        pad_pk = jnp.full((16,), n_real + (half << 14), jnp.int32)

        def fill(i, _):
            cpk_v[pl.ds(i * 16, 16)] = pad_pk
            return 0
        lax.fori_loop(0, cap // 16, fill, 0)

        def body(i, pos):
            sv = src_v[pl.ds(i * 16, 16)]
            dv = dst_v[pl.ds(i * 16, 16)]
            owned = (dv >= lo) & (dv < lo + half)
            ow = jnp.where(owned, jnp.int32(1), jnp.int32(0))
            pk = sv | ((dv - lo) << 14)
            for j in range(16):
                cpk_v[pl.ds(pos, 16)] = jnp.broadcast_to(pk[j], (16,))
                pos = pos + ow[j]
            return pos

        cnt = lax.fori_loop(0, epw // 16, body, jnp.int32(0))
        cpk_v[pl.ds(cnt, 16)] = pad_pk

        def unpack(i, _):
            v = cpk_v[pl.ds(i * 16, 16)]
            src_v[pl.ds(i * 16, 16)] = v & jnp.int32(0x3FFF)
            dst_v[pl.ds(i * 16, 16)] = lax.shift_right_logical(v, 14)
            return 0
        lax.fori_loop(0, epw // 16, unpack, 0)

        cnt_v[pl.ds(0, 16)] = jnp.broadcast_to(cnt, (16,)).astype(jnp.int32)
        pltpu.sync_copy(cnt_v, ocnt_hbm.at[c, s])
        pltpu.sync_copy(src_v, osrc_hbm.at[c, pl.ds(s * epw, epw)])
        pltpu.sync_copy(dst_v, odst_hbm.at[c, pl.ds(s * epw, epw)])

    return part_kernel


def _make_spmm_kernel(e_pad, n_pad, d):
    half = n_pad // NC
    j_per_w = e_pad // (NS * CH)
    rpt = half // NS

    K = 2

    @functools.partial(
        pl.kernel,
        out_type=jax.ShapeDtypeStruct((NC, half, d), jnp.float32),
        mesh=_sc_mesh(),
        scratch_types=[
            pltpu.VMEM((j_per_w, CH), jnp.int32),
            pltpu.VMEM((j_per_w, CH), jnp.int32),
            pltpu.VMEM((16,), jnp.int32),
            [pltpu.VMEM((CH, d), jnp.float32) for _ in range(K)],
            [pltpu.SemaphoreType.DMA for _ in range(K)],
            [pltpu.SemaphoreType.DMA for _ in range(K)],
            pltpu.VMEM_SHARED((half + 8, d), jnp.float32),
        ],
    )
    def spmm_kernel(g_hbm, src_hbm, dst_hbm, cnt_hbm, out_hbm,
                    src_v, dst_v, cnt_v, rows, gsem, ssem, acc_sh):
        c = lax.axis_index("c")
        s = lax.axis_index("s")

        def zrow(i, _):
            for cc in range(d // 16):
                rows[0][i, pl.ds(cc * 16, 16)] = jnp.zeros((16,), jnp.float32)
            return 0
        lax.fori_loop(0, CH, zrow, 0)
        nz = rpt // CH
        for k in range(nz):
            pltpu.sync_copy(rows[0], acc_sh.at[pl.ds(s * rpt + k * CH, CH)])
        if rpt % CH:
            pltpu.sync_copy(rows[0].at[pl.ds(0, rpt % CH)],
                            acc_sh.at[pl.ds(s * rpt + nz * CH, rpt % CH)])
        plsc.subcore_barrier()

        pltpu.sync_copy(src_hbm.at[c, pl.ds(s * j_per_w, j_per_w)], src_v)
        pltpu.sync_copy(dst_hbm.at[c, pl.ds(s * j_per_w, j_per_w)], dst_v)
        pltpu.sync_copy(cnt_hbm.at[c, s], cnt_v)

        cnt = cnt_v[pl.ds(0, 16)][0]
        n_grp = lax.max((cnt + CH * K - 1) // (CH * K), jnp.int32(1))

        for r in range(K):
            pltpu.async_copy(g_hbm.at[src_v.at[r]], rows[r], gsem[r])

        def group(g, _):
            base = g * K
            for r in range(K):
                pltpu.make_async_copy(g_hbm.at[src_v.at[base + r]],
                                      rows[r], gsem[r]).wait()
                pltpu.async_copy(rows[r], acc_sh.at[dst_v.at[base + r]],
                                 ssem[r], add=True)

            @pl.when(g + 1 < n_grp)
            def _():
                for r in range(K):
                    pltpu.make_async_copy(
                        rows[r], acc_sh.at[dst_v.at[base + r]],
                        ssem[r]).wait()
                    pltpu.async_copy(g_hbm.at[src_v.at[base + K + r]],
                                     rows[r], gsem[r])
            return 0

        lax.fori_loop(0, n_grp, group, 0)
        last = (n_grp - 1) * K
        for r in range(K):
            pltpu.make_async_copy(rows[r], acc_sh.at[dst_v.at[last + r]],
                                  ssem[r]).wait()

        plsc.subcore_barrier()
        pltpu.sync_copy(acc_sh.at[pl.ds(s * rpt, rpt)],
                        out_hbm.at[c, pl.ds(s * rpt, rpt)])

    return spmm_kernel


def _tc1_body(cnt_ref, x_ref, w_ref, dinv_ref, g_ref):
    deg = cnt_ref[:, 0:1] + cnt_ref[:, 1:2] + 1.0
    dinv = lax.rsqrt(deg)
    dinv_ref[...] = dinv
    h = jnp.dot(x_ref[...], w_ref[...], preferred_element_type=jnp.float32)
    g_ref[...] = h * dinv


def _tc2_body(acc_ref, g1_ref, dinv_ref, b_ref, w_ref, g2_ref):
    a = acc_ref[...] + g1_ref[...]
    dinv = dinv_ref[...]
    h = jnp.maximum(dinv * a + b_ref[...], 0.0)
    g2_ref[...] = dinv * jnp.dot(h, w_ref[...],
                                 preferred_element_type=jnp.float32)


def _tc3_body(acc_ref, g2_ref, dinv_ref, b_ref, out_ref):
    a = acc_ref[...] + g2_ref[...]
    out_ref[...] = dinv_ref[...] * a + b_ref[...]


def kernel(x, edge_index, edge_attr, W1, b1, W2, b2):
    n, d = x.shape
    e = edge_index.shape[1]

    n_pad = (n + 1 + 255) // 256 * 256
    half = n_pad // NC
    e_pad = (e + NS * CH * 8 - 1) // (NS * CH * 8) * (NS * CH * 8)

    src = edge_index[0]
    dst = edge_index[1]
    pad = e_pad - e
    src_p = jnp.concatenate([src, jnp.full((pad,), n, jnp.int32)])
    dst_p = jnp.concatenate([dst, jnp.full((pad,), n, jnp.int32)])
    dst2d = dst_p.reshape(e_pad // CH, CH)

    x_pad = jnp.zeros((n_pad, d), x.dtype).at[:n].set(x)
    b1r = b1.reshape(1, d)
    b2r = b2.reshape(1, d)

    deg_k = _make_deg_kernel(e_pad, n_pad)
    part_k = _make_part_kernel(e_pad, n_pad, n)
    spmm_k = _make_spmm_kernel(e_pad, n_pad, d)

    psrc, pdst, pcnt = part_k(src_p, dst_p)
    psrc3 = psrc.reshape(NC, e_pad // CH, CH)
    pdst3 = pdst.reshape(NC, e_pad // CH, CH)

    BISECT_JNP_DEG = False
    if BISECT_JNP_DEG:
        cnt0 = jnp.zeros((n_pad,), jnp.float32).at[dst].add(1.0)
        cnt_t = jnp.stack([cnt0, jnp.zeros_like(cnt0)], axis=1)
    else:
        cnt = deg_k(dst2d)
        cnt_t = cnt.T

    r = 1280
    grid = n_pad // r
    row_spec = pl.BlockSpec((r, d), lambda i: (i, 0))
    col_spec = pl.BlockSpec((r, 1), lambda i: (i, 0))
    full_spec = pl.BlockSpec((d, d), lambda i: (0, 0))
    bias_spec = pl.BlockSpec((1, d), lambda i: (0, 0))

    dinv, g1 = pl.pallas_call(
        _tc1_body,
        grid=grid,
        in_specs=[pl.BlockSpec((r, NC), lambda i: (i, 0)),
                  row_spec, full_spec],
        out_specs=[col_spec, row_spec],
        out_shape=[jax.ShapeDtypeStruct((n_pad, 1), jnp.float32),
                   jax.ShapeDtypeStruct((n_pad, d), jnp.float32)],
    )(cnt_t, x_pad, W1)

    acc1 = spmm_k(g1, psrc3, pdst3, pcnt).reshape(n_pad, d)

    g2 = pl.pallas_call(
        _tc2_body,
        grid=grid,
        in_specs=[row_spec, row_spec, col_spec, bias_spec, full_spec],
        out_specs=row_spec,
        out_shape=jax.ShapeDtypeStruct((n_pad, d), jnp.float32),
    )(acc1, g1, dinv, b1r, W2)

    acc2 = spmm_k(g2, psrc3, pdst3, pcnt).reshape(n_pad, d)

    out = pl.pallas_call(
        _tc3_body,
        grid=grid,
        in_specs=[row_spec, row_spec, col_spec, bias_spec],
        out_specs=row_spec,
        out_shape=jax.ShapeDtypeStruct((n_pad, d), jnp.float32),
    )(acc2, g2, dinv, b2r)

    return out[:n]


# speedup vs baseline: 2.4319x; 1.9919x over previous
import functools
import jax
import jax.numpy as jnp
from jax import lax
from jax.experimental import pallas as pl
from jax.experimental.pallas import tpu as pltpu
from jax.experimental.pallas import tpu_sc as plsc

NC = 2
NS = 16
NW = NC * NS
CH = 128


def _sc_mesh():
    return plsc.VectorSubcoreMesh(core_axis_name="c", subcore_axis_name="s")


def _make_deg_kernel(e_pad, n_pad):
    j_per_w = e_pad // (NW * CH)
    rpt = n_pad // NS
    dw = 16

    @functools.partial(
        pl.kernel,
        out_type=jax.ShapeDtypeStruct((NC, n_pad), jnp.float32),
        mesh=_sc_mesh(),
        scratch_types=[
            pltpu.VMEM((j_per_w, CH), jnp.int32),
            pltpu.VMEM((CH,), jnp.float32),
            pltpu.VMEM((rpt,), jnp.float32),
            pltpu.VMEM_SHARED((n_pad + 16,), jnp.float32),
        ],
    )
    def deg_kernel(dst_hbm, out_hbm, dst_v, ones_v, zbuf, deg_sh):
        c = lax.axis_index("c")
        s = lax.axis_index("s")
        w = c * NS + s

        ones = jnp.ones((16,), jnp.float32)
        zeros = jnp.zeros((16,), jnp.float32)

        for i in range(CH // 16):
            ones_v[pl.ds(i * 16, 16)] = ones

        def fill_zero(i, _):
            zbuf[pl.ds(i * 16, 16)] = zeros
            return 0
        lax.fori_loop(0, rpt // 16, fill_zero, 0)

        pltpu.sync_copy(zbuf, deg_sh.at[pl.ds(s * rpt, rpt)])
        pltpu.sync_copy(dst_hbm.at[pl.ds(w * j_per_w, j_per_w)], dst_v)
        plsc.subcore_barrier()

        def body(j, _):
            pltpu.sync_copy(ones_v, deg_sh.at[dst_v.at[j]], add=True)
            return 0
        lax.fori_loop(0, j_per_w, body, 0)

        plsc.subcore_barrier()
        pltpu.sync_copy(deg_sh.at[pl.ds(s * rpt, rpt)],
                        out_hbm.at[c, pl.ds(s * rpt, rpt)])

    return deg_kernel


def _make_part_kernel(e_pad, n_pad, n_real):
    half = n_pad // NC
    jw = e_pad // (NS * CH)
    epw = e_pad // NS
    cap = epw + 16

    @functools.partial(
        pl.kernel,
        out_type=[
            jax.ShapeDtypeStruct((NC, NS * epw), jnp.int32),
            jax.ShapeDtypeStruct((NC, NS * epw), jnp.int32),
            jax.ShapeDtypeStruct((NC, NS, 16), jnp.int32),
        ],
        mesh=_sc_mesh(),
        scratch_types=[
            pltpu.VMEM((epw,), jnp.int32),
            pltpu.VMEM((epw,), jnp.int32),
            pltpu.VMEM((cap,), jnp.int32),
            pltpu.VMEM((16,), jnp.int32),
        ],
    )
    def part_kernel(src_hbm, dst_hbm, osrc_hbm, odst_hbm, ocnt_hbm,
                    src_v, dst_v, cpk_v, cnt_v):
        c = lax.axis_index("c")
        s = lax.axis_index("s")
        lo = c * half

        pltpu.sync_copy(src_hbm.at[pl.ds(s * epw, epw)], src_v)
        pltpu.sync_copy(dst_hbm.at[pl.ds(s * epw, epw)], dst_v)

        pad_pk = jnp.full((16,), n_real + (half << 14), jnp.int32)

        def fill(i, _):
            cpk_v[pl.ds(i * 16, 16)] = pad_pk
            return 0
        lax.fori_loop(0, cap // 16, fill, 0)

        def body(i, pos):
            sv = src_v[pl.ds(i * 16, 16)]
            dv = dst_v[pl.ds(i * 16, 16)]
            owned = (dv >= lo) & (dv < lo + half)
            ow = jnp.where(owned, jnp.int32(1), jnp.int32(0))
            pk = sv | ((dv - lo) << 14)
            for j in range(16):
                cpk_v[pl.ds(pos, 16)] = jnp.broadcast_to(pk[j], (16,))
                pos = pos + ow[j]
            return pos

        cnt = lax.fori_loop(0, epw // 16, body, jnp.int32(0))
        cpk_v[pl.ds(cnt, 16)] = pad_pk

        def unpack(i, _):
            v = cpk_v[pl.ds(i * 16, 16)]
            src_v[pl.ds(i * 16, 16)] = v & jnp.int32(0x3FFF)
            dst_v[pl.ds(i * 16, 16)] = lax.shift_right_logical(v, 14)
            return 0
        lax.fori_loop(0, epw // 16, unpack, 0)

        cnt_v[pl.ds(0, 16)] = jnp.broadcast_to(cnt, (16,)).astype(jnp.int32)
        pltpu.sync_copy(cnt_v, ocnt_hbm.at[c, s])
        pltpu.sync_copy(src_v, osrc_hbm.at[c, pl.ds(s * epw, epw)])
        pltpu.sync_copy(dst_v, odst_hbm.at[c, pl.ds(s * epw, epw)])

    return part_kernel


def _make_spmm_kernel(e_pad, n_pad, d):
    half = n_pad // NC
    j_per_w = e_pad // (NS * CH)
    rpt = half // NS

    K = 2

    @functools.partial(
        pl.kernel,
        out_type=jax.ShapeDtypeStruct((NC, half, d), jnp.float32),
        mesh=_sc_mesh(),
        scratch_types=[
            pltpu.VMEM((j_per_w, CH), jnp.int32),
            pltpu.VMEM((j_per_w, CH), jnp.int32),
            pltpu.VMEM((16,), jnp.int32),
            [pltpu.VMEM((CH, d), jnp.float32) for _ in range(K)],
            [pltpu.SemaphoreType.DMA for _ in range(K)],
            [pltpu.SemaphoreType.DMA for _ in range(K)],
            pltpu.VMEM_SHARED((half + 8, d), jnp.float32),
        ],
    )
    def spmm_kernel(g_hbm, src_hbm, dst_hbm, cnt_hbm, out_hbm,
                    src_v, dst_v, cnt_v, rows, gsem, ssem, acc_sh):
        c = lax.axis_index("c")
        s = lax.axis_index("s")

        def zrow(i, _):
            for cc in range(d // 16):
                rows[0][i, pl.ds(cc * 16, 16)] = jnp.zeros((16,), jnp.float32)
            return 0
        lax.fori_loop(0, CH, zrow, 0)
        nz = rpt // CH
        for k in range(nz):
            pltpu.sync_copy(rows[0], acc_sh.at[pl.ds(s * rpt + k * CH, CH)])
        if rpt % CH:
            pltpu.sync_copy(rows[0].at[pl.ds(0, rpt % CH)],
                            acc_sh.at[pl.ds(s * rpt + nz * CH, rpt % CH)])
        plsc.subcore_barrier()

        pltpu.sync_copy(src_hbm.at[c, pl.ds(s * j_per_w, j_per_w)], src_v)
        pltpu.sync_copy(dst_hbm.at[c, pl.ds(s * j_per_w, j_per_w)], dst_v)
        pltpu.sync_copy(cnt_hbm.at[c, s], cnt_v)

        cnt = cnt_v[pl.ds(0, 16)][0]
        n_grp = lax.max((cnt + CH * K - 1) // (CH * K), jnp.int32(1))

        for r in range(K):
            pltpu.async_copy(g_hbm.at[src_v.at[r]], rows[r], gsem[r])

        def group(g, _):
            base = g * K
            for r in range(K):
                pltpu.make_async_copy(g_hbm.at[src_v.at[base + r]],
                                      rows[r], gsem[r]).wait()
                pltpu.async_copy(rows[r], acc_sh.at[dst_v.at[base + r]],
                                 ssem[r], add=True)

            @pl.when(g + 1 < n_grp)
            def _():
                for r in range(K):
                    pltpu.make_async_copy(
                        rows[r], acc_sh.at[dst_v.at[base + r]],
                        ssem[r]).wait()
                    pltpu.async_copy(g_hbm.at[src_v.at[base + K + r]],
                                     rows[r], gsem[r])
            return 0

        lax.fori_loop(0, n_grp, group, 0)
        last = (n_grp - 1) * K
        for r in range(K):
            pltpu.make_async_copy(rows[r], acc_sh.at[dst_v.at[last + r]],
                                  ssem[r]).wait()

        plsc.subcore_barrier()
        pltpu.sync_copy(acc_sh.at[pl.ds(s * rpt, rpt)],
                        out_hbm.at[c, pl.ds(s * rpt, rpt)])

    return spmm_kernel


def _tc1_body(cnt_ref, x_ref, w_ref, dinv_ref, g_ref):
    deg = cnt_ref[:, 0:1] + cnt_ref[:, 1:2] + 1.0
    dinv = lax.rsqrt(deg)
    dinv_ref[...] = dinv
    h = jnp.dot(x_ref[...], w_ref[...], preferred_element_type=jnp.float32)
    g_ref[...] = h * dinv


def _tc2_body(acc_ref, g1_ref, dinv_ref, b_ref, w_ref, g2_ref):
    a = acc_ref[...] + g1_ref[...]
    dinv = dinv_ref[...]
    h = jnp.maximum(dinv * a + b_ref[...], 0.0)
    g2_ref[...] = dinv * jnp.dot(h, w_ref[...],
                                 preferred_element_type=jnp.float32)


def _tc3_body(acc_ref, g2_ref, dinv_ref, b_ref, out_ref):
    a = acc_ref[...] + g2_ref[...]
    out_ref[...] = dinv_ref[...] * a + b_ref[...]


def kernel(x, edge_index, edge_attr, W1, b1, W2, b2):
    n, d = x.shape
    e = edge_index.shape[1]

    n_pad = (n + 1 + 255) // 256 * 256
    half = n_pad // NC
    e_pad = (e + NS * CH * 8 - 1) // (NS * CH * 8) * (NS * CH * 8)

    src = edge_index[0]
    dst = edge_index[1]
    pad = e_pad - e
    src_p = jnp.concatenate([src, jnp.full((pad,), n, jnp.int32)])
    dst_p = jnp.concatenate([dst, jnp.full((pad,), n_pad, jnp.int32)])
    dst2d = dst_p.reshape(e_pad // CH, CH)

    x_pad = jnp.zeros((n_pad, d), x.dtype).at[:n].set(x)
    b1r = b1.reshape(1, d)
    b2r = b2.reshape(1, d)

    deg_k = _make_deg_kernel(e_pad, n_pad)
    part_k = _make_part_kernel(e_pad, n_pad, n)
    spmm_k = _make_spmm_kernel(e_pad, n_pad, d)

    psrc, pdst, pcnt = part_k(src_p, dst_p)
    psrc3 = psrc.reshape(NC, e_pad // CH, CH)
    pdst3 = pdst.reshape(NC, e_pad // CH, CH)

    BISECT_JNP_DEG = False
    if BISECT_JNP_DEG:
        cnt0 = jnp.zeros((n_pad,), jnp.float32).at[dst].add(1.0)
        cnt_t = jnp.stack([cnt0, jnp.zeros_like(cnt0)], axis=1)
    else:
        cnt = deg_k(dst2d)
        cnt_t = cnt.T

    r = 1280
    grid = n_pad // r
    row_spec = pl.BlockSpec((r, d), lambda i: (i, 0))
    col_spec = pl.BlockSpec((r, 1), lambda i: (i, 0))
    full_spec = pl.BlockSpec((d, d), lambda i: (0, 0))
    bias_spec = pl.BlockSpec((1, d), lambda i: (0, 0))

    dinv, g1 = pl.pallas_call(
        _tc1_body,
        grid=grid,
        in_specs=[pl.BlockSpec((r, NC), lambda i: (i, 0)),
                  row_spec, full_spec],
        out_specs=[col_spec, row_spec],
        out_shape=[jax.ShapeDtypeStruct((n_pad, 1), jnp.float32),
                   jax.ShapeDtypeStruct((n_pad, d), jnp.float32)],
    )(cnt_t, x_pad, W1)

    acc1 = spmm_k(g1, psrc3, pdst3, pcnt).reshape(n_pad, d)

    g2 = pl.pallas_call(
        _tc2_body,
        grid=grid,
        in_specs=[row_spec, row_spec, col_spec, bias_spec, full_spec],
        out_specs=row_spec,
        out_shape=jax.ShapeDtypeStruct((n_pad, d), jnp.float32),
    )(acc1, g1, dinv, b1r, W2)

    acc2 = spmm_k(g2, psrc3, pdst3, pcnt).reshape(n_pad, d)

    out = pl.pallas_call(
        _tc3_body,
        grid=grid,
        in_specs=[row_spec, row_spec, col_spec, bias_spec],
        out_specs=row_spec,
        out_shape=jax.ShapeDtypeStruct((n_pad, d), jnp.float32),
    )(acc2, g2, dinv, b2r)

    return out[:n]


# speedup vs baseline: 2.5087x; 1.0316x over previous
import functools
import jax
import jax.numpy as jnp
from jax import lax
from jax.experimental import pallas as pl
from jax.experimental.pallas import tpu as pltpu
from jax.experimental.pallas import tpu_sc as plsc

NC = 2
NS = 16
NW = NC * NS
CH = 128


def _sc_mesh():
    return plsc.VectorSubcoreMesh(core_axis_name="c", subcore_axis_name="s")


def _make_deg_kernel(e_pad, n_pad):
    j_per_w = e_pad // (NW * CH)
    rpt = n_pad // NS
    dw = 16

    @functools.partial(
        pl.kernel,
        out_type=jax.ShapeDtypeStruct((NC, n_pad), jnp.float32),
        mesh=_sc_mesh(),
        scratch_types=[
            pltpu.VMEM((j_per_w, CH), jnp.int32),
            pltpu.VMEM((CH,), jnp.float32),
            pltpu.VMEM((rpt,), jnp.float32),
            pltpu.VMEM_SHARED((n_pad + 16,), jnp.float32),
        ],
    )
    def deg_kernel(dst_hbm, out_hbm, dst_v, ones_v, zbuf, deg_sh):
        c = lax.axis_index("c")
        s = lax.axis_index("s")
        w = c * NS + s

        ones = jnp.ones((16,), jnp.float32)
        zeros = jnp.zeros((16,), jnp.float32)

        for i in range(CH // 16):
            ones_v[pl.ds(i * 16, 16)] = ones

        def fill_zero(i, _):
            zbuf[pl.ds(i * 16, 16)] = zeros
            return 0
        lax.fori_loop(0, rpt // 16, fill_zero, 0)

        pltpu.sync_copy(zbuf, deg_sh.at[pl.ds(s * rpt, rpt)])
        pltpu.sync_copy(dst_hbm.at[pl.ds(w * j_per_w, j_per_w)], dst_v)
        plsc.subcore_barrier()

        def body(j, _):
            pltpu.sync_copy(ones_v, deg_sh.at[dst_v.at[j]], add=True)
            return 0
        lax.fori_loop(0, j_per_w, body, 0)

        plsc.subcore_barrier()
        pltpu.sync_copy(deg_sh.at[pl.ds(s * rpt, rpt)],
                        out_hbm.at[c, pl.ds(s * rpt, rpt)])

    return deg_kernel


def _make_part_kernel(e_pad, n_pad, n_real):
    half = n_pad // NC
    jw = e_pad // (NS * CH)
    epw = e_pad // NS
    cap = epw + 16

    @functools.partial(
        pl.kernel,
        out_type=[
            jax.ShapeDtypeStruct((NC, NS * epw), jnp.int32),
            jax.ShapeDtypeStruct((NC, NS * epw), jnp.int32),
            jax.ShapeDtypeStruct((NC, NS, 16), jnp.int32),
        ],
        mesh=_sc_mesh(),
        scratch_types=[
            pltpu.VMEM((epw,), jnp.int32),
            pltpu.VMEM((epw,), jnp.int32),
            pltpu.VMEM((cap,), jnp.int32),
            pltpu.VMEM((16,), jnp.int32),
        ],
    )
    def part_kernel(src_hbm, dst_hbm, osrc_hbm, odst_hbm, ocnt_hbm,
                    src_v, dst_v, cpk_v, cnt_v):
        c = lax.axis_index("c")
        s = lax.axis_index("s")
        lo = c * half

        pltpu.sync_copy(src_hbm.at[pl.ds(s * epw, epw)], src_v)
        pltpu.sync_copy(dst_hbm.at[pl.ds(s * epw, epw)], dst_v)

        pad_pk = jnp.full((16,), n_real + (half << 14), jnp.int32)

        def fill(i, _):
            cpk_v[pl.ds(i * 16, 16)] = pad_pk
            return 0
        lax.fori_loop(0, cap // 16, fill, 0)

        def body(i, pos):
            sv = src_v[pl.ds(i * 16, 16)]
            dv = dst_v[pl.ds(i * 16, 16)]
            owned = (dv >= lo) & (dv < lo + half)
            ow = jnp.where(owned, jnp.int32(1), jnp.int32(0))
            pk = sv | ((dv - lo) << 14)
            for j in range(16):
                cpk_v[pl.ds(pos, 16)] = jnp.broadcast_to(pk[j], (16,))
                pos = pos + ow[j]
            return pos

        cnt = lax.fori_loop(0, epw // 16, body, jnp.int32(0))
        cpk_v[pl.ds(cnt, 16)] = pad_pk

        def unpack(i, _):
            v = cpk_v[pl.ds(i * 16, 16)]
            src_v[pl.ds(i * 16, 16)] = v & jnp.int32(0x3FFF)
            dst_v[pl.ds(i * 16, 16)] = lax.shift_right_logical(v, 14)
            return 0
        lax.fori_loop(0, epw // 16, unpack, 0)

        cnt_v[pl.ds(0, 16)] = jnp.broadcast_to(cnt, (16,)).astype(jnp.int32)
        pltpu.sync_copy(cnt_v, ocnt_hbm.at[c, s])
        pltpu.sync_copy(src_v, osrc_hbm.at[c, pl.ds(s * epw, epw)])
        pltpu.sync_copy(dst_v, odst_hbm.at[c, pl.ds(s * epw, epw)])

    return part_kernel


def _make_spmm_kernel(e_pad, n_pad, d):
    half = n_pad // NC
    j_per_w = e_pad // (NS * CH)
    rpt = half // NS

    K = 2

    @functools.partial(
        pl.kernel,
        out_type=jax.ShapeDtypeStruct((NC, half, d), jnp.float32),
        mesh=_sc_mesh(),
        scratch_types=[
            pltpu.VMEM((j_per_w, CH), jnp.int32),
            pltpu.VMEM((j_per_w, CH), jnp.int32),
            pltpu.VMEM((16,), jnp.int32),
            [pltpu.VMEM((CH, d), jnp.float32) for _ in range(K)],
            [pltpu.SemaphoreType.DMA for _ in range(K)],
            [pltpu.SemaphoreType.DMA for _ in range(K)],
            pltpu.VMEM_SHARED((half + 8, d), jnp.float32),
        ],
    )
    def spmm_kernel(g_hbm, src_hbm, dst_hbm, cnt_hbm, out_hbm,
                    src_v, dst_v, cnt_v, rows, gsem, ssem, acc_sh):
        c = lax.axis_index("c")
        s = lax.axis_index("s")

        def zrow(i, _):
            for cc in range(d // 16):
                rows[0][i, pl.ds(cc * 16, 16)] = jnp.zeros((16,), jnp.float32)
            return 0
        lax.fori_loop(0, CH, zrow, 0)
        nz = rpt // CH
        for k in range(nz):
            pltpu.sync_copy(rows[0], acc_sh.at[pl.ds(s * rpt + k * CH, CH)])
        if rpt % CH:
            pltpu.sync_copy(rows[0].at[pl.ds(0, rpt % CH)],
                            acc_sh.at[pl.ds(s * rpt + nz * CH, rpt % CH)])
        plsc.subcore_barrier()

        pltpu.sync_copy(src_hbm.at[c, pl.ds(s * j_per_w, j_per_w)], src_v)
        pltpu.sync_copy(dst_hbm.at[c, pl.ds(s * j_per_w, j_per_w)], dst_v)
        pltpu.sync_copy(cnt_hbm.at[c, s], cnt_v)

        cnt = cnt_v[pl.ds(0, 16)][0]
        n_grp = lax.max((cnt + CH * K - 1) // (CH * K), jnp.int32(1))

        m = 2 * n_grp

        def gather(j, r):
            pltpu.async_copy(g_hbm.at[src_v.at[j]], rows[r], gsem[r])

        def gwait(j, r):
            pltpu.make_async_copy(g_hbm.at[src_v.at[j]],
                                  rows[r], gsem[r]).wait()

        def scat(j, r):
            pltpu.async_copy(rows[r], acc_sh.at[dst_v.at[j]],
                             ssem[r], add=True)

        def swait(j, r):
            pltpu.make_async_copy(rows[r], acc_sh.at[dst_v.at[j]],
                                  ssem[r]).wait()

        gather(0, 0)
        gwait(0, 0)
        scat(0, 0)
        gather(1, 1)

        def pair(g, _):
            j1 = 2 * g + 1
            gwait(j1, 1)
            scat(j1, 1)

            @pl.when(j1 + 1 < m)
            def _():
                swait(j1 - 1, 0)
                gather(j1 + 1, 0)
                gwait(j1 + 1, 0)
                scat(j1 + 1, 0)

                @pl.when(j1 + 2 < m)
                def _():
                    swait(j1, 1)
                    gather(j1 + 2, 1)
            return 0

        lax.fori_loop(0, n_grp, pair, 0)
        swait(m - 2, 0)
        swait(m - 1, 1)

        plsc.subcore_barrier()
        pltpu.sync_copy(acc_sh.at[pl.ds(s * rpt, rpt)],
                        out_hbm.at[c, pl.ds(s * rpt, rpt)])

    return spmm_kernel


def _tc1_body(cnt_ref, x_ref, w_ref, dinv_ref, g_ref):
    deg = cnt_ref[:, 0:1] + cnt_ref[:, 1:2] + 1.0
    dinv = lax.rsqrt(deg)
    dinv_ref[...] = dinv
    h = jnp.dot(x_ref[...], w_ref[...], preferred_element_type=jnp.float32)
    g_ref[...] = h * dinv


def _tc2_body(acc_ref, g1_ref, dinv_ref, b_ref, w_ref, g2_ref):
    a = acc_ref[...] + g1_ref[...]
    dinv = dinv_ref[...]
    h = jnp.maximum(dinv * a + b_ref[...], 0.0)
    g2_ref[...] = dinv * jnp.dot(h, w_ref[...],
                                 preferred_element_type=jnp.float32)


def _tc3_body(acc_ref, g2_ref, dinv_ref, b_ref, out_ref):
    a = acc_ref[...] + g2_ref[...]
    out_ref[...] = dinv_ref[...] * a + b_ref[...]


def kernel(x, edge_index, edge_attr, W1, b1, W2, b2):
    n, d = x.shape
    e = edge_index.shape[1]

    n_pad = (n + 1 + 255) // 256 * 256
    half = n_pad // NC
    e_pad = (e + NS * CH * 8 - 1) // (NS * CH * 8) * (NS * CH * 8)

    src = edge_index[0]
    dst = edge_index[1]
    pad = e_pad - e
    src_p = jnp.concatenate([src, jnp.full((pad,), n, jnp.int32)])
    dst_p = jnp.concatenate([dst, jnp.full((pad,), n_pad, jnp.int32)])
    dst2d = dst_p.reshape(e_pad // CH, CH)

    x_pad = jnp.zeros((n_pad, d), x.dtype).at[:n].set(x)
    b1r = b1.reshape(1, d)
    b2r = b2.reshape(1, d)

    deg_k = _make_deg_kernel(e_pad, n_pad)
    part_k = _make_part_kernel(e_pad, n_pad, n)
    spmm_k = _make_spmm_kernel(e_pad, n_pad, d)

    psrc, pdst, pcnt = part_k(src_p, dst_p)
    psrc3 = psrc.reshape(NC, e_pad // CH, CH)
    pdst3 = pdst.reshape(NC, e_pad // CH, CH)

    BISECT_JNP_DEG = False
    if BISECT_JNP_DEG:
        cnt0 = jnp.zeros((n_pad,), jnp.float32).at[dst].add(1.0)
        cnt_t = jnp.stack([cnt0, jnp.zeros_like(cnt0)], axis=1)
    else:
        cnt = deg_k(dst2d)
        cnt_t = cnt.T

    r = 1280
    grid = n_pad // r
    row_spec = pl.BlockSpec((r, d), lambda i: (i, 0))
    col_spec = pl.BlockSpec((r, 1), lambda i: (i, 0))
    full_spec = pl.BlockSpec((d, d), lambda i: (0, 0))
    bias_spec = pl.BlockSpec((1, d), lambda i: (0, 0))

    dinv, g1 = pl.pallas_call(
        _tc1_body,
        grid=grid,
        in_specs=[pl.BlockSpec((r, NC), lambda i: (i, 0)),
                  row_spec, full_spec],
        out_specs=[col_spec, row_spec],
        out_shape=[jax.ShapeDtypeStruct((n_pad, 1), jnp.float32),
                   jax.ShapeDtypeStruct((n_pad, d), jnp.float32)],
    )(cnt_t, x_pad, W1)

    acc1 = spmm_k(g1, psrc3, pdst3, pcnt).reshape(n_pad, d)

    g2 = pl.pallas_call(
        _tc2_body,
        grid=grid,
        in_specs=[row_spec, row_spec, col_spec, bias_spec, full_spec],
        out_specs=row_spec,
        out_shape=jax.ShapeDtypeStruct((n_pad, d), jnp.float32),
    )(acc1, g1, dinv, b1r, W2)

    acc2 = spmm_k(g2, psrc3, pdst3, pcnt).reshape(n_pad, d)

    out = pl.pallas_call(
        _tc3_body,
        grid=grid,
        in_specs=[row_spec, row_spec, col_spec, bias_spec],
        out_specs=row_spec,
        out_shape=jax.ShapeDtypeStruct((n_pad, d), jnp.float32),
    )(acc2, g2, dinv, b2r)

    return out[:n]


# speedup vs baseline: 2.5665x; 1.0230x over previous
import functools
import jax
import jax.numpy as jnp
from jax import lax
from jax.experimental import pallas as pl
from jax.experimental.pallas import tpu as pltpu
from jax.experimental.pallas import tpu_sc as plsc

NC = 2
NS = 16
NW = NC * NS
CH = 128


def _sc_mesh():
    return plsc.VectorSubcoreMesh(core_axis_name="c", subcore_axis_name="s")


def _make_part_kernel(e_pad, n_pad, n_real):
    half = n_pad // NC
    jw = e_pad // (NS * CH)
    epw = e_pad // NS
    cap = epw + 16

    hjw = jw // NC
    rpt = n_pad // NS

    @functools.partial(
        pl.kernel,
        out_type=[
            jax.ShapeDtypeStruct((NC, NS * jw, CH), jnp.int32),
            jax.ShapeDtypeStruct((NC, NS * jw, CH), jnp.int32),
            jax.ShapeDtypeStruct((NC, NS, 16), jnp.int32),
            jax.ShapeDtypeStruct((NC, n_pad), jnp.float32),
        ],
        mesh=_sc_mesh(),
        scratch_types=[
            pltpu.VMEM((jw, CH), jnp.int32),
            pltpu.VMEM((jw, CH), jnp.int32),
            pltpu.VMEM((cap,), jnp.int32),
            pltpu.VMEM((16,), jnp.int32),
            pltpu.VMEM((CH,), jnp.float32),
            pltpu.VMEM((rpt,), jnp.float32),
            pltpu.VMEM_SHARED((n_pad + 16,), jnp.float32),
            pltpu.SemaphoreType.DMA,
        ],
    )
    def part_kernel(src_hbm, dst_hbm, osrc_hbm, odst_hbm, ocnt_hbm, odeg_hbm,
                    src_v, dst_v, cpk_v, cnt_v, ones_v, zdeg_v, deg_sh, dsem):
        c = lax.axis_index("c")
        s = lax.axis_index("s")
        lo = c * half

        pltpu.sync_copy(src_hbm.at[pl.ds(s * jw, jw)], src_v)
        pltpu.sync_copy(dst_hbm.at[pl.ds(s * jw, jw)], dst_v)

        ones = jnp.ones((16,), jnp.float32)
        zeros = jnp.zeros((16,), jnp.float32)
        for i in range(CH // 16):
            ones_v[pl.ds(i * 16, 16)] = ones

        def zfill(i, _):
            zdeg_v[pl.ds(i * 16, 16)] = zeros
            return 0
        lax.fori_loop(0, rpt // 16, zfill, 0)
        pltpu.sync_copy(zdeg_v, deg_sh.at[pl.ds(s * rpt, rpt)])
        plsc.subcore_barrier()

        def dfire(j, _):
            pltpu.async_copy(ones_v, deg_sh.at[dst_v.at[c * hjw + j]],
                             dsem, add=True)
            return 0
        lax.fori_loop(0, hjw, dfire, 0)

        pad_pk = jnp.full((16,), n_real + (half << 14), jnp.int32)

        def fill(i, _):
            cpk_v[pl.ds(i * 16, 16)] = pad_pk
            return 0
        lax.fori_loop(0, cap // 16, fill, 0)

        def body(row, pos):
            for seg in range(CH // 16):
                sv = src_v[row, pl.ds(seg * 16, 16)]
                dv = dst_v[row, pl.ds(seg * 16, 16)]
                owned = (dv >= lo) & (dv < lo + half)
                ow = jnp.where(owned, jnp.int32(1), jnp.int32(0))
                pk = sv | ((dv - lo) << 14)
                for j in range(16):
                    cpk_v[pl.ds(pos, 16)] = jnp.broadcast_to(pk[j], (16,))
                    pos = pos + ow[j]
            return pos

        cnt = lax.fori_loop(0, jw, body, jnp.int32(0))
        cpk_v[pl.ds(cnt, 16)] = pad_pk

        def ddrain(j, _):
            pltpu.make_async_copy(ones_v, deg_sh.at[dst_v.at[c * hjw + j]],
                                  dsem).wait()
            return 0
        lax.fori_loop(0, hjw, ddrain, 0)

        def unpack(row, _):
            for seg in range(CH // 16):
                v = cpk_v[pl.ds((row * (CH // 16) + seg) * 16, 16)]
                src_v[row, pl.ds(seg * 16, 16)] = v & jnp.int32(0x3FFF)
                dst_v[row, pl.ds(seg * 16, 16)] = lax.shift_right_logical(v, 14)
            return 0
        lax.fori_loop(0, jw, unpack, 0)

        cnt_v[pl.ds(0, 16)] = jnp.broadcast_to(cnt, (16,)).astype(jnp.int32)
        plsc.subcore_barrier()
        pltpu.sync_copy(cnt_v, ocnt_hbm.at[c, s])
        pltpu.sync_copy(src_v, osrc_hbm.at[c, pl.ds(s * jw, jw)])
        pltpu.sync_copy(dst_v, odst_hbm.at[c, pl.ds(s * jw, jw)])
        pltpu.sync_copy(deg_sh.at[pl.ds(s * rpt, rpt)],
                        odeg_hbm.at[c, pl.ds(s * rpt, rpt)])

    return part_kernel


def _make_spmm_kernel(e_pad, n_pad, d):
    half = n_pad // NC
    j_per_w = e_pad // (NS * CH)
    rpt = half // NS

    K = 2

    @functools.partial(
        pl.kernel,
        out_type=jax.ShapeDtypeStruct((NC, half, d), jnp.float32),
        mesh=_sc_mesh(),
        scratch_types=[
            pltpu.VMEM((j_per_w, CH), jnp.int32),
            pltpu.VMEM((j_per_w, CH), jnp.int32),
            pltpu.VMEM((16,), jnp.int32),
            [pltpu.VMEM((CH, d), jnp.float32) for _ in range(K)],
            [pltpu.SemaphoreType.DMA for _ in range(K)],
            [pltpu.SemaphoreType.DMA for _ in range(K)],
            pltpu.VMEM_SHARED((half + 8, d), jnp.float32),
        ],
    )
    def spmm_kernel(g_hbm, src_hbm, dst_hbm, cnt_hbm, out_hbm,
                    src_v, dst_v, cnt_v, rows, gsem, ssem, acc_sh):
        c = lax.axis_index("c")
        s = lax.axis_index("s")

        def zrow(i, _):
            for cc in range(d // 16):
                rows[0][i, pl.ds(cc * 16, 16)] = jnp.zeros((16,), jnp.float32)
            return 0
        lax.fori_loop(0, CH, zrow, 0)
        nz = rpt // CH
        for k in range(nz):
            pltpu.sync_copy(rows[0], acc_sh.at[pl.ds(s * rpt + k * CH, CH)])
        if rpt % CH:
            pltpu.sync_copy(rows[0].at[pl.ds(0, rpt % CH)],
                            acc_sh.at[pl.ds(s * rpt + nz * CH, rpt % CH)])
        plsc.subcore_barrier()

        pltpu.sync_copy(src_hbm.at[c, pl.ds(s * j_per_w, j_per_w)], src_v)
        pltpu.sync_copy(dst_hbm.at[c, pl.ds(s * j_per_w, j_per_w)], dst_v)
        pltpu.sync_copy(cnt_hbm.at[c, s], cnt_v)

        cnt = cnt_v[pl.ds(0, 16)][0]
        n_grp = lax.max((cnt + CH * K - 1) // (CH * K), jnp.int32(1))

        m = 2 * n_grp

        def gather(j, r):
            pltpu.async_copy(g_hbm.at[src_v.at[j]], rows[r], gsem[r])

        def gwait(j, r):
            pltpu.make_async_copy(g_hbm.at[src_v.at[j]],
                                  rows[r], gsem[r]).wait()

        def scat(j, r):
            pltpu.async_copy(rows[r], acc_sh.at[dst_v.at[j]],
                             ssem[r], add=True)

        def swait(j, r):
            pltpu.make_async_copy(rows[r], acc_sh.at[dst_v.at[j]],
                                  ssem[r]).wait()

        gather(0, 0)
        gwait(0, 0)
        scat(0, 0)
        gather(1, 1)

        def pair(g, _):
            j1 = 2 * g + 1
            gwait(j1, 1)
            scat(j1, 1)

            @pl.when(j1 + 1 < m)
            def _():
                swait(j1 - 1, 0)
                gather(j1 + 1, 0)
                gwait(j1 + 1, 0)
                scat(j1 + 1, 0)

                @pl.when(j1 + 2 < m)
                def _():
                    swait(j1, 1)
                    gather(j1 + 2, 1)
            return 0

        lax.fori_loop(0, n_grp, pair, 0)
        swait(m - 2, 0)
        swait(m - 1, 1)

        plsc.subcore_barrier()
        pltpu.sync_copy(acc_sh.at[pl.ds(s * rpt, rpt)],
                        out_hbm.at[c, pl.ds(s * rpt, rpt)])

    return spmm_kernel


def _tc1_body(cnt_ref, x_ref, w_ref, dinv_ref, g_ref):
    deg = cnt_ref[:, 0:1] + cnt_ref[:, 1:2] + 1.0
    dinv = lax.rsqrt(deg)
    dinv_ref[...] = dinv
    h = jnp.dot(x_ref[...], w_ref[...], preferred_element_type=jnp.float32)
    g_ref[...] = h * dinv


def _tc2_body(acc_ref, g1_ref, dinv_ref, b_ref, w_ref, g2_ref):
    a = acc_ref[...] + g1_ref[...]
    dinv = dinv_ref[...]
    h = jnp.maximum(dinv * a + b_ref[...], 0.0)
    g2_ref[...] = dinv * jnp.dot(h, w_ref[...],
                                 preferred_element_type=jnp.float32)


def _tc3_body(acc_ref, g2_ref, dinv_ref, b_ref, out_ref):
    a = acc_ref[...] + g2_ref[...]
    out_ref[...] = dinv_ref[...] * a + b_ref[...]


def kernel(x, edge_index, edge_attr, W1, b1, W2, b2):
    n, d = x.shape
    e = edge_index.shape[1]

    n_pad = (n + 1 + 255) // 256 * 256
    half = n_pad // NC
    e_pad = (e + NS * CH * 8 - 1) // (NS * CH * 8) * (NS * CH * 8)

    src = edge_index[0]
    dst = edge_index[1]
    pad = e_pad - e
    src_p = jnp.concatenate([src, jnp.full((pad,), n, jnp.int32)])
    dst_p = jnp.concatenate([dst, jnp.full((pad,), n_pad, jnp.int32)])
    dst2d = dst_p.reshape(e_pad // CH, CH)

    x_pad = jnp.zeros((n_pad, d), x.dtype).at[:n].set(x)
    b1r = b1.reshape(1, d)
    b2r = b2.reshape(1, d)

    part_k = _make_part_kernel(e_pad, n_pad, n)
    spmm_k = _make_spmm_kernel(e_pad, n_pad, d)

    src2d = src_p.reshape(e_pad // CH, CH)
    psrc3, pdst3, pcnt, pdeg = part_k(src2d, dst2d)
    cnt_t = pdeg.T

    r = 1280
    grid = n_pad // r
    row_spec = pl.BlockSpec((r, d), lambda i: (i, 0))
    col_spec = pl.BlockSpec((r, 1), lambda i: (i, 0))
    full_spec = pl.BlockSpec((d, d), lambda i: (0, 0))
    bias_spec = pl.BlockSpec((1, d), lambda i: (0, 0))

    dinv, g1 = pl.pallas_call(
        _tc1_body,
        grid=grid,
        in_specs=[pl.BlockSpec((r, NC), lambda i: (i, 0)),
                  row_spec, full_spec],
        out_specs=[col_spec, row_spec],
        out_shape=[jax.ShapeDtypeStruct((n_pad, 1), jnp.float32),
                   jax.ShapeDtypeStruct((n_pad, d), jnp.float32)],
    )(cnt_t, x_pad, W1)

    acc1 = spmm_k(g1, psrc3, pdst3, pcnt).reshape(n_pad, d)

    g2 = pl.pallas_call(
        _tc2_body,
        grid=grid,
        in_specs=[row_spec, row_spec, col_spec, bias_spec, full_spec],
        out_specs=row_spec,
        out_shape=jax.ShapeDtypeStruct((n_pad, d), jnp.float32),
    )(acc1, g1, dinv, b1r, W2)

    acc2 = spmm_k(g2, psrc3, pdst3, pcnt).reshape(n_pad, d)

    out = pl.pallas_call(
        _tc3_body,
        grid=grid,
        in_specs=[row_spec, row_spec, col_spec, bias_spec],
        out_specs=row_spec,
        out_shape=jax.ShapeDtypeStruct((n_pad, d), jnp.float32),
    )(acc2, g2, dinv, b2r)

    return out[:n]
